# Initial kernel scaffold; baseline (speedup 1.0000x reference)
import jax
import jax.numpy as jnp
from jax.experimental import pallas as pl


def kernel(x, edge_index, W_T, W_A, ln_gamma, ln_beta):
    raise NotImplementedError("write your pallas kernel here")


# speedup vs baseline: 4.4009x; 4.4009x over previous
import functools

import jax
import jax.numpy as jnp
from jax import lax
from jax.experimental import pallas as pl
from jax.experimental.pallas import tpu as pltpu
from jax.experimental.pallas import tpu_sc as plsc

N = 10000
D = 128
E = 320000

NC = 2
NS = 16
NW = NC * NS
EPT = E // NW
K = 40
CH = EPT // K
NP = 10240
RPT = NP // NS
ZR = 32
AR = NP // D


def _pre_body(x_ref, wt_ref, wa_ref, u_ref):
    x = x_ref[...]
    t = jnp.dot(x, wt_ref[...], preferred_element_type=jnp.float32)
    u_ref[...] = jnp.concatenate([t * wa_ref[...], x], axis=1)


def _sc_body(u_hbm, x_hbm, src_hbm, dst_hbm, out_feat, out_att,
             srcv, dstv, uv, xv, mv, zv, av2, attv, ididx, sem1, sem2,
             acc_sh, att_sh):
    c = lax.axis_index("c")
    s = lax.axis_index("s")
    wid = c * NS + s

    def zrow(i, _):
        for j in range(D // 16):
            zv[i, pl.ds(j * 16, 16)] = jnp.zeros((16,), jnp.float32)
        return 0
    lax.fori_loop(0, ZR, zrow, 0)

    def zatt(i, _):
        for j in range(D // 16):
            attv[i, pl.ds(j * 16, 16)] = jnp.zeros((16,), jnp.float32)
        return 0
    lax.fori_loop(0, AR, zatt, 0)

    for j in range(AR // 16):
        ididx[pl.ds(j * 16, 16)] = lax.iota(jnp.int32, 16) + j * 16

    for t in range(RPT // ZR):
        pltpu.sync_copy(zv, acc_sh.at[pl.ds(s * RPT + t * ZR, ZR)])

    @pl.when(s < AR // 8)
    def _():
        pltpu.sync_copy(zv.at[pl.ds(0, 8)], att_sh.at[pl.ds(s * 8, 8)])

    plsc.subcore_barrier()

    def chunk(ch, _):
        base = wid * EPT + ch * K
        pltpu.sync_copy(src_hbm.at[pl.ds(base, K)], srcv)
        pltpu.sync_copy(dst_hbm.at[pl.ds(base, K)], dstv)
        cp1 = pltpu.async_copy(u_hbm.at[srcv], uv, sem1)
        cp2 = pltpu.async_copy(x_hbm.at[dstv], xv, sem2)
        cp1.wait()
        cp2.wait()

        def edge(k, _):
            dot = uv[k, pl.ds(0, 16)] * xv[k, pl.ds(0, 16)]
            for j in range(1, D // 16):
                dot += uv[k, pl.ds(j * 16, 16)] * xv[k, pl.ds(j * 16, 16)]
            tot = jnp.sum(dot)
            a = jnp.exp(jnp.broadcast_to(tot, (16,)))
            for j in range(D // 16):
                mv[k, pl.ds(j * 16, 16)] = a * uv[k, pl.ds(D + j * 16, 16)]
            av2[k, pl.ds(0, 16)] = a
            return 0
        lax.fori_loop(0, K, edge, 0)

        for g in range(K // 16):
            ei = lax.iota(jnp.int32, 16) + g * 16
            vals = plsc.load_gather(av2, [ei, jnp.zeros((16,), jnp.int32)])
            d16 = dstv[pl.ds(g * 16, 16)]
            plsc.addupdate_scatter(attv, [d16 // D, d16 % D], vals)

        pltpu.sync_copy(mv, acc_sh.at[dstv], add=True)
        return 0
    lax.fori_loop(0, CH, chunk, 0)

    pltpu.sync_copy(attv, att_sh.at[ididx], add=True)
    plsc.subcore_barrier()

    for t in range(RPT // ZR):
        pltpu.sync_copy(acc_sh.at[pl.ds(s * RPT + t * ZR, ZR)], zv)
        pltpu.sync_copy(zv, out_feat.at[pl.ds(c * NP + s * RPT + t * ZR, ZR)])

    @pl.when(s < AR // 8)
    def _():
        pltpu.sync_copy(att_sh.at[pl.ds(s * 8, 8)],
                        out_att.at[pl.ds(c * AR + s * 8, 8)])


def _post_body(p0_ref, p1_ref, a0_ref, a1_ref, g_ref, b_ref, o_ref):
    acc = p0_ref[...] + p1_ref[...]
    ssum = a0_ref[...] + a1_ref[...]
    ssum = jnp.where(ssum > 0.0, ssum, 1.0)
    h = jnp.maximum(acc / ssum, 0.0)
    mu = jnp.mean(h, axis=1, keepdims=True)
    var = jnp.mean((h - mu) * (h - mu), axis=1, keepdims=True)
    o_ref[...] = (h - mu) * lax.rsqrt(var + 1e-5) * g_ref[...] + b_ref[...]


def kernel(x, edge_index, W_T, W_A, ln_gamma, ln_beta):
    wa_row = W_A.reshape(1, D)

    rb = 1000
    u_full = pl.pallas_call(
        _pre_body,
        grid=(N // rb,),
        in_specs=[
            pl.BlockSpec((rb, D), lambda i: (i, 0)),
            pl.BlockSpec((D, D), lambda i: (0, 0)),
            pl.BlockSpec((1, D), lambda i: (0, 0)),
        ],
        out_specs=pl.BlockSpec((rb, 2 * D), lambda i: (i, 0)),
        out_shape=jax.ShapeDtypeStruct((N, 2 * D), jnp.float32),
    )(x, W_T, wa_row)

    src = edge_index[0]
    dst = edge_index[1]

    mesh = plsc.VectorSubcoreMesh(
        core_axis_name="c", subcore_axis_name="s",
        num_cores=NC, num_subcores=NS)
    sc_call = functools.partial(
        pl.kernel,
        out_type=(jax.ShapeDtypeStruct((NC * NP, D), jnp.float32),
                  jax.ShapeDtypeStruct((NC * AR, D), jnp.float32)),
        mesh=mesh,
        compiler_params=pltpu.CompilerParams(needs_layout_passes=False),
        scratch_types=[
            pltpu.VMEM((K,), jnp.int32),
            pltpu.VMEM((K,), jnp.int32),
            pltpu.VMEM((K, 2 * D), jnp.float32),
            pltpu.VMEM((K, D), jnp.float32),
            pltpu.VMEM((K, D), jnp.float32),
            pltpu.VMEM((ZR, D), jnp.float32),
            pltpu.VMEM((K, 16), jnp.float32),
            pltpu.VMEM((AR, D), jnp.float32),
            pltpu.VMEM((AR,), jnp.int32),
            pltpu.SemaphoreType.DMA,
            pltpu.SemaphoreType.DMA,
            pltpu.VMEM_SHARED((NP, D), jnp.float32),
            pltpu.VMEM_SHARED((AR, D), jnp.float32),
        ],
    )(_sc_body)
    feats, atts = sc_call(u_full, x, src, dst)

    att0 = atts[:AR].reshape(-1)[:N].reshape(N, 1)
    att1 = atts[AR:].reshape(-1)[:N].reshape(N, 1)

    gam = ln_gamma.reshape(1, D)
    bet = ln_beta.reshape(1, D)
    h = pl.pallas_call(
        _post_body,
        grid=(N // rb,),
        in_specs=[
            pl.BlockSpec((rb, D), lambda i: (i, 0)),
            pl.BlockSpec((rb, D), lambda i: (i, 0)),
            pl.BlockSpec((rb, 1), lambda i: (i, 0)),
            pl.BlockSpec((rb, 1), lambda i: (i, 0)),
            pl.BlockSpec((1, D), lambda i: (0, 0)),
            pl.BlockSpec((1, D), lambda i: (0, 0)),
        ],
        out_specs=pl.BlockSpec((rb, D), lambda i: (i, 0)),
        out_shape=jax.ShapeDtypeStruct((N, D), jnp.float32),
    )(feats[:N], feats[NP:NP + N], att0, att1, gam, bet)
    return h
